# baseline (speedup 1.0000x reference)
import functools

import jax
import jax.numpy as jnp
from jax.experimental import pallas as pl

B, D_IN, C, NC = 16384, 500, 64, 2
BLK = 2048


def _leaky(x):
    return jnp.where(x >= 0, x, 0.01 * x)


def _fused_kernel(texts_ref, labels_ref, w1_ref, b1_ref, w2_ref, b2_ref,
                  w3_ref, b3_ref, w4_ref, b4_ref,
                  logits_ref, preds_ref, proj_ref, loss_ref):
    i = pl.program_id(0)

    x = texts_ref[...]
    h = jnp.dot(x, w1_ref[...], preferred_element_type=jnp.float32)
    h = _leaky(h + b1_ref[...])
    proj = jnp.dot(h, w2_ref[...], preferred_element_type=jnp.float32)
    proj = proj + b2_ref[...]
    proj_ref[...] = proj

    c = jnp.dot(proj, w3_ref[...], preferred_element_type=jnp.float32)
    c = _leaky(c + b3_ref[...])
    lr = jnp.dot(c, w4_ref[...], preferred_element_type=jnp.float32)
    lr = lr + b4_ref[...]

    m = jnp.max(lr, axis=1, keepdims=True)
    e = jnp.exp(lr - m)
    s = jnp.sum(e, axis=1, keepdims=True)
    logits_ref[...] = e / s

    lr0 = lr[:, 0:1]
    lr1 = lr[:, 1:2]
    preds_ref[...] = (lr1 > lr0).astype(jnp.int32)

    logp = lr - m - jnp.log(s)
    lab = labels_ref[...]
    picked = jnp.where(lab == 0, logp[:, 0:1], logp[:, 1:2])
    partial = (jnp.sum(picked) * (-1.0 / B)).reshape(1, 1)

    @pl.when(i == 0)
    def _():
        loss_ref[...] = jnp.zeros((1, 1), jnp.float32)

    loss_ref[...] += partial


@functools.partial(jax.jit, static_argnames=())
def kernel(texts, img, labels, W1, b1, W2, b2, W3, b3, W4, b4):
    del img
    grid = (B // BLK,)
    labels2 = labels.reshape(B, 1).astype(jnp.int32)

    out_shapes = (
        jax.ShapeDtypeStruct((B, NC), jnp.float32),
        jax.ShapeDtypeStruct((B, 1), jnp.int32),
        jax.ShapeDtypeStruct((B, C), jnp.float32),
        jax.ShapeDtypeStruct((1, 1), jnp.float32),
    )

    full = lambda *dims: pl.BlockSpec(dims, lambda i: (0,) * len(dims))
    in_specs = [
        pl.BlockSpec((BLK, D_IN), lambda i: (i, 0)),
        pl.BlockSpec((BLK, 1), lambda i: (i, 0)),
        full(D_IN, 2 * C),
        full(1, 2 * C),
        full(2 * C, C),
        full(1, C),
        full(C, C),
        full(1, C),
        full(C, NC),
        full(1, NC),
    ]
    out_specs = (
        pl.BlockSpec((BLK, NC), lambda i: (i, 0)),
        pl.BlockSpec((BLK, 1), lambda i: (i, 0)),
        pl.BlockSpec((BLK, C), lambda i: (i, 0)),
        pl.BlockSpec((1, 1), lambda i: (0, 0)),
    )

    logits, preds2, projections, loss_sum = pl.pallas_call(
        _fused_kernel,
        grid=grid,
        in_specs=in_specs,
        out_specs=out_specs,
        out_shape=out_shapes,
    )(texts, labels2, W1, b1.reshape(1, -1), W2, b2.reshape(1, -1),
      W3, b3.reshape(1, -1), W4, b4.reshape(1, -1))

    preds = preds2.reshape(B)
    vectors = projections[:, None, :]
    loss = loss_sum.reshape(())
    return (logits, preds, projections, vectors, loss)


# speedup vs baseline: 1.9903x; 1.9903x over previous
import functools

import jax
import jax.numpy as jnp
from jax.experimental import pallas as pl

B, D_IN, C, NC = 16384, 500, 64, 2
BLK = 2048


def _leaky(x):
    return jnp.where(x >= 0, x, 0.01 * x)


def _fused_kernel(texts_ref, labels_ref, w1_ref, b1_ref, w2_ref, b2_ref,
                  w3t_ref, b3c_ref, w4t_ref, b4c_ref,
                  logits_t_ref, preds_t_ref, proj_ref, loss_ref):
    i = pl.program_id(0)

    x = texts_ref[...]
    h = jnp.dot(x, w1_ref[...], preferred_element_type=jnp.float32)
    h = _leaky(h + b1_ref[...])
    proj = jnp.dot(h, w2_ref[...], preferred_element_type=jnp.float32)
    proj = proj + b2_ref[...]
    proj_ref[...] = proj

    proj_t = proj.T
    c_t = jnp.dot(w3t_ref[...], proj_t, preferred_element_type=jnp.float32)
    c_t = _leaky(c_t + b3c_ref[...])
    lr_t = jnp.dot(w4t_ref[...], c_t, preferred_element_type=jnp.float32)
    lr_t = lr_t + b4c_ref[...]

    lr0 = lr_t[0:1, :]
    lr1 = lr_t[1:2, :]
    m = jnp.maximum(lr0, lr1)
    e0 = jnp.exp(lr0 - m)
    e1 = jnp.exp(lr1 - m)
    s = e0 + e1
    inv_s = 1.0 / s
    logits_t_ref[...] = jnp.concatenate([e0 * inv_s, e1 * inv_s], axis=0)

    preds_t_ref[...] = (lr1 > lr0).astype(jnp.int32)

    log_s = jnp.log(s)
    logp0 = lr0 - m - log_s
    logp1 = lr1 - m - log_s
    lab = labels_ref[...]
    picked = jnp.where(lab == 0, logp0, logp1)
    partial = (jnp.sum(picked) * (-1.0 / B)).reshape(1, 1)

    @pl.when(i == 0)
    def _():
        loss_ref[...] = jnp.zeros((1, 1), jnp.float32)

    loss_ref[...] += partial


@functools.partial(jax.jit, static_argnames=())
def kernel(texts, img, labels, W1, b1, W2, b2, W3, b3, W4, b4):
    del img
    grid = (B // BLK,)
    labels2 = labels.reshape(1, B).astype(jnp.int32)

    out_shapes = (
        jax.ShapeDtypeStruct((NC, B), jnp.float32),
        jax.ShapeDtypeStruct((1, B), jnp.int32),
        jax.ShapeDtypeStruct((B, C), jnp.float32),
        jax.ShapeDtypeStruct((1, 1), jnp.float32),
    )

    full = lambda *dims: pl.BlockSpec(dims, lambda i: (0,) * len(dims))
    in_specs = [
        pl.BlockSpec((BLK, D_IN), lambda i: (i, 0)),
        pl.BlockSpec((1, BLK), lambda i: (0, i)),
        full(D_IN, 2 * C),
        full(1, 2 * C),
        full(2 * C, C),
        full(1, C),
        full(C, C),
        full(C, 1),
        full(NC, C),
        full(NC, 1),
    ]
    out_specs = (
        pl.BlockSpec((NC, BLK), lambda i: (0, i)),
        pl.BlockSpec((1, BLK), lambda i: (0, i)),
        pl.BlockSpec((BLK, C), lambda i: (i, 0)),
        pl.BlockSpec((1, 1), lambda i: (0, 0)),
    )

    logits_t, preds_t, projections, loss_sum = pl.pallas_call(
        _fused_kernel,
        grid=grid,
        in_specs=in_specs,
        out_specs=out_specs,
        out_shape=out_shapes,
    )(texts, labels2, W1, b1.reshape(1, -1), W2, b2.reshape(1, -1),
      W3.T, b3.reshape(-1, 1), W4.T, b4.reshape(-1, 1))

    logits = logits_t.T
    preds = preds_t.reshape(B)
    vectors = projections[:, None, :]
    loss = loss_sum.reshape(())
    return (logits, preds, projections, vectors, loss)


# speedup vs baseline: 7.4758x; 3.7560x over previous
import functools

import jax
import jax.numpy as jnp
from jax import lax
from jax.experimental import pallas as pl

B, D_IN, C, NC = 16384, 500, 64, 2
BLK = 2048

_CONTRACT_D0 = (((0,), (0,)), ((), ()))


def _leaky(x):
    return jnp.where(x >= 0, x, 0.01 * x)


def _fused_kernel(xt_ref, labels_ref, w1_ref, b1_ref, w2t_ref, b2_ref,
                  w3_ref, b3_ref, w4t_ref, b4_ref,
                  logits_t_ref, preds_t_ref, proj_t_ref, vec_t_ref, loss_ref):
    i = pl.program_id(0)

    xt = xt_ref[...]
    h_t = lax.dot_general(w1_ref[...], xt, _CONTRACT_D0,
                          preferred_element_type=jnp.float32)
    h_t = _leaky(h_t + b1_ref[...].T)
    proj_t = jnp.dot(w2t_ref[...], h_t,
                     preferred_element_type=jnp.float32)
    proj_t = proj_t + b2_ref[...].T
    proj_t_ref[...] = proj_t
    vec_t_ref[...] = proj_t

    c_t = lax.dot_general(w3_ref[...], proj_t, _CONTRACT_D0,
                          preferred_element_type=jnp.float32)
    c_t = _leaky(c_t + b3_ref[...].T)
    lr_t = jnp.dot(w4t_ref[...], c_t,
                   preferred_element_type=jnp.float32)
    lr_t = lr_t + b4_ref[...].T

    lr0 = lr_t[0:1, :]
    lr1 = lr_t[1:2, :]
    m = jnp.maximum(lr0, lr1)
    e0 = jnp.exp(lr0 - m)
    e1 = jnp.exp(lr1 - m)
    s = e0 + e1
    inv_s = 1.0 / s
    logits_t_ref[...] = jnp.concatenate([e0 * inv_s, e1 * inv_s], axis=0)

    preds_t_ref[...] = (lr1 > lr0).astype(jnp.int32)

    log_s = jnp.log(s)
    logp0 = lr0 - m - log_s
    logp1 = lr1 - m - log_s
    lab = labels_ref[...]
    picked = jnp.where(lab == 0, logp0, logp1)
    partial = (jnp.sum(picked) * (-1.0 / B)).reshape(1, 1)

    @pl.when(i == 0)
    def _():
        loss_ref[...] = jnp.zeros((1, 1), jnp.float32)

    loss_ref[...] += partial


@functools.partial(jax.jit, static_argnames=())
def kernel(texts, img, labels, W1, b1, W2, b2, W3, b3, W4, b4):
    del img
    grid = (B // BLK,)
    texts_t = texts.T
    labels2 = labels.reshape(1, B).astype(jnp.int32)

    out_shapes = (
        jax.ShapeDtypeStruct((NC, B), jnp.float32),
        jax.ShapeDtypeStruct((1, B), jnp.int32),
        jax.ShapeDtypeStruct((C, B), jnp.float32),
        jax.ShapeDtypeStruct((C, B), jnp.float32),
        jax.ShapeDtypeStruct((1, 1), jnp.float32),
    )

    full = lambda *dims: pl.BlockSpec(dims, lambda i: (0,) * len(dims))
    in_specs = [
        pl.BlockSpec((D_IN, BLK), lambda i: (0, i)),
        pl.BlockSpec((1, BLK), lambda i: (0, i)),
        full(D_IN, 2 * C),
        full(1, 2 * C),
        full(C, 2 * C),
        full(1, C),
        full(C, C),
        full(1, C),
        full(NC, C),
        full(1, NC),
    ]
    out_specs = (
        pl.BlockSpec((NC, BLK), lambda i: (0, i)),
        pl.BlockSpec((1, BLK), lambda i: (0, i)),
        pl.BlockSpec((C, BLK), lambda i: (0, i)),
        pl.BlockSpec((C, BLK), lambda i: (0, i)),
        pl.BlockSpec((1, 1), lambda i: (0, 0)),
    )

    logits_t, preds_t, proj_t, vec_t, loss_sum = pl.pallas_call(
        _fused_kernel,
        grid=grid,
        in_specs=in_specs,
        out_specs=out_specs,
        out_shape=out_shapes,
    )(texts_t, labels2, W1, b1.reshape(1, -1), W2.T, b2.reshape(1, -1),
      W3, b3.reshape(1, -1), W4.T, b4.reshape(1, -1))

    logits = logits_t.T
    preds = preds_t.reshape(B)
    projections = proj_t.T
    vectors = vec_t.T[:, None, :]
    loss = loss_sum.reshape(())
    return (logits, preds, projections, vectors, loss)


# speedup vs baseline: 8.5171x; 1.1393x over previous
import functools

import jax
import jax.numpy as jnp
from jax import lax
from jax.experimental import pallas as pl

B, D_IN, C, NC = 16384, 500, 64, 2
BLK = 4096

_CONTRACT_D0 = (((0,), (0,)), ((), ()))


def _leaky(x):
    return jnp.where(x >= 0, x, 0.01 * x)


def _fused_kernel(xt_ref, labels_ref, w1_ref, b1_ref, w2t_ref, b2_ref,
                  w3_ref, b3_ref, w4t_ref, b4_ref,
                  logits_t_ref, preds_t_ref, proj_t_ref, vec_t_ref, loss_ref):
    i = pl.program_id(0)

    xt = xt_ref[...]
    h_t = lax.dot_general(w1_ref[...], xt, _CONTRACT_D0,
                          preferred_element_type=jnp.float32)
    h_t = _leaky(h_t + b1_ref[...].T)
    proj_t = jnp.dot(w2t_ref[...], h_t,
                     preferred_element_type=jnp.float32)
    proj_t = proj_t + b2_ref[...].T
    proj_t_ref[...] = proj_t
    vec_t_ref[...] = proj_t

    c_t = lax.dot_general(w3_ref[...], proj_t, _CONTRACT_D0,
                          preferred_element_type=jnp.float32)
    c_t = _leaky(c_t + b3_ref[...].T)
    lr_t = jnp.dot(w4t_ref[...], c_t,
                   preferred_element_type=jnp.float32)
    lr_t = lr_t + b4_ref[...].T

    lr0 = lr_t[0:1, :]
    lr1 = lr_t[1:2, :]
    m = jnp.maximum(lr0, lr1)
    e0 = jnp.exp(lr0 - m)
    e1 = jnp.exp(lr1 - m)
    s = e0 + e1
    inv_s = 1.0 / s
    logits_t_ref[...] = jnp.concatenate([e0 * inv_s, e1 * inv_s], axis=0)

    preds_t_ref[...] = (lr1 > lr0).astype(jnp.int32)

    log_s = jnp.log(s)
    logp0 = lr0 - m - log_s
    logp1 = lr1 - m - log_s
    lab = labels_ref[...]
    picked = jnp.where(lab == 0, logp0, logp1)
    partial = (jnp.sum(picked) * (-1.0 / B)).reshape(1, 1)

    @pl.when(i == 0)
    def _():
        loss_ref[...] = jnp.zeros((1, 1), jnp.float32)

    loss_ref[...] += partial


@functools.partial(jax.jit, static_argnames=())
def kernel(texts, img, labels, W1, b1, W2, b2, W3, b3, W4, b4):
    del img
    grid = (B // BLK,)
    texts_t = texts.T
    labels2 = labels.reshape(1, B).astype(jnp.int32)

    out_shapes = (
        jax.ShapeDtypeStruct((NC, B), jnp.float32),
        jax.ShapeDtypeStruct((1, B), jnp.int32),
        jax.ShapeDtypeStruct((C, B), jnp.float32),
        jax.ShapeDtypeStruct((C, B), jnp.float32),
        jax.ShapeDtypeStruct((1, 1), jnp.float32),
    )

    full = lambda *dims: pl.BlockSpec(dims, lambda i: (0,) * len(dims))
    in_specs = [
        pl.BlockSpec((D_IN, BLK), lambda i: (0, i)),
        pl.BlockSpec((1, BLK), lambda i: (0, i)),
        full(D_IN, 2 * C),
        full(1, 2 * C),
        full(C, 2 * C),
        full(1, C),
        full(C, C),
        full(1, C),
        full(NC, C),
        full(1, NC),
    ]
    out_specs = (
        pl.BlockSpec((NC, BLK), lambda i: (0, i)),
        pl.BlockSpec((1, BLK), lambda i: (0, i)),
        pl.BlockSpec((C, BLK), lambda i: (0, i)),
        pl.BlockSpec((C, BLK), lambda i: (0, i)),
        pl.BlockSpec((1, 1), lambda i: (0, 0)),
    )

    logits_t, preds_t, proj_t, vec_t, loss_sum = pl.pallas_call(
        _fused_kernel,
        grid=grid,
        in_specs=in_specs,
        out_specs=out_specs,
        out_shape=out_shapes,
    )(texts_t, labels2, W1, b1.reshape(1, -1), W2.T, b2.reshape(1, -1),
      W3, b3.reshape(1, -1), W4.T, b4.reshape(1, -1))

    logits = logits_t.T
    preds = preds_t.reshape(B)
    projections = proj_t.T
    vectors = vec_t.T[:, None, :]
    loss = loss_sum.reshape(())
    return (logits, preds, projections, vectors, loss)
